# baseline (speedup 1.0000x reference)
import functools

import jax
import jax.numpy as jnp
from jax import lax
from jax.experimental import pallas as pl
from jax.experimental.pallas import tpu as pltpu
from jax.experimental.pallas import tpu_sc as plsc

EMBED_DIM = 16
PACK = 8
NUM_SC_CORES = 2
NUM_SC_SUBCORES = 16
NUM_WORKERS = NUM_SC_CORES * NUM_SC_SUBCORES
NBUF = 2
CHUNK = 256


def _linearize_body(tbl3, stg_hbm, n_groups_pad, sem, wid):
    gpw = n_groups_pad // NUM_WORKERS
    extra = n_groups_pad - gpw * NUM_WORKERS
    base_g = wid * gpw
    window = 8

    def fire(g):
        for s in range(2):
            pltpu.async_copy(tbl3.at[s, :, pl.ds(g * 128, 128)],
                             stg_hbm.at[s, g], sem)

    def wait_one():
        pltpu.make_async_copy(stg_hbm.at[0, 0], stg_hbm.at[1, 0], sem).wait()
        pltpu.make_async_copy(stg_hbm.at[0, 0], stg_hbm.at[1, 0], sem).wait()

    @pl.loop(0, gpw)
    def _(i):
        fire(base_g + i)

        @pl.when(i >= window)
        def _():
            wait_one()

    for _ in range(window):
        wait_one()

    @pl.when(wid < extra)
    def _():
        fire(gpw * NUM_WORKERS + wid)
        wait_one()


def _sc_linearize(mt3, ut3, gm_pad, gu_pad):
    mesh = plsc.VectorSubcoreMesh(core_axis_name="c", subcore_axis_name="s")

    @functools.partial(
        pl.kernel,
        mesh=mesh,
        out_type=(
            jax.ShapeDtypeStruct((2, gm_pad, 8, 128), jnp.float32),
            jax.ShapeDtypeStruct((2, gu_pad, 8, 128), jnp.float32),
        ),
        scratch_types=[
            pltpu.SemaphoreType.DMA,
            pltpu.SemaphoreType.DMA,
        ],
        compiler_params=pltpu.CompilerParams(use_tc_tiling_on_sc=True),
    )
    def lin_kernel(mt_hbm, ut_hbm, ms_hbm, us_hbm, sem_m, sem_u):
        wid = lax.axis_index("s") * NUM_SC_CORES + lax.axis_index("c")
        _linearize_body(ut_hbm, us_hbm, gu_pad, sem_u, wid)
        _linearize_body(mt_hbm, ms_hbm, gm_pad, sem_m, wid)

    return lin_kernel(mt3, ut3)


def _repack_body(stg, out_hbm, n_rows, asm_v, in_sems, out_sems, wid):
    n_groups = n_rows // 128
    tail = n_rows - n_groups * 128
    gpw = n_groups // NUM_WORKERS
    extra = n_groups - gpw * NUM_WORKERS
    base_g = wid * gpw
    nbuf = asm_v.shape[0]

    def fire_in(b, g):
        for s in range(2):
            for j in range(8):
                pltpu.async_copy(
                    stg.at[s, g, j],
                    asm_v.at[b, pl.ds(0, 16), pl.ds(64 * s + 8 * j, 8)],
                    in_sems[b])

    def wait_in(b):
        pltpu.make_async_copy(out_hbm.at[pl.ds(0, 16), :], asm_v.at[b],
                              in_sems[b]).wait()

    def wait_out(b):
        pltpu.make_async_copy(out_hbm.at[pl.ds(0, 16), :], asm_v.at[b],
                              out_sems[b]).wait()

    for b in range(nbuf):
        fire_in(b, base_g + b)

    @pl.loop(0, gpw, step=nbuf)
    def _(i0):
        for b in range(nbuf):
            g = base_g + i0 + b
            wait_in(b)
            pltpu.async_copy(asm_v.at[b], out_hbm.at[pl.ds(16 * g, 16), :],
                             out_sems[b])
        for b in range(nbuf):
            wait_out(b)
            g_next = base_g + i0 + b + nbuf

            @pl.when(g_next < base_g + gpw)
            def _():
                fire_in(b, g_next)

    @pl.when(wid < extra)
    def _():
        g = gpw * NUM_WORKERS + wid
        fire_in(0, g)
        wait_in(0)
        pltpu.async_copy(asm_v.at[0], out_hbm.at[pl.ds(16 * g, 16), :],
                         out_sems[0])
        wait_out(0)

    if tail:
        @pl.when(wid == extra)
        def _():
            g = n_groups
            for s in range(2):
                for j in range(8):
                    pltpu.sync_copy(
                        stg.at[s, g, j, pl.ds(0, tail // 8), :],
                        asm_v.at[0, pl.ds(0, tail // 8),
                                 pl.ds(64 * s + 8 * j, 8)])
            pltpu.sync_copy(asm_v.at[0, pl.ds(0, tail // 8), :],
                            out_hbm.at[pl.ds(16 * g, tail // 8), :])


def _sc_repack(stg_m, stg_u, n_m, n_u):
    mesh = plsc.VectorSubcoreMesh(core_axis_name="c", subcore_axis_name="s")

    @functools.partial(
        pl.kernel,
        mesh=mesh,
        out_type=(
            jax.ShapeDtypeStruct((n_m // PACK, 128), jnp.float32),
            jax.ShapeDtypeStruct((n_u // PACK, 128), jnp.float32),
        ),
        scratch_types=(
            [pltpu.VMEM((NBUF, 16, 128), jnp.float32)]
            + [pltpu.SemaphoreType.DMA] * (2 * NBUF)
        ),
        compiler_params=pltpu.CompilerParams(use_tc_tiling_on_sc=False),
    )
    def repack_kernel(ms_hbm, us_hbm, mp_hbm, up_hbm, asm_v, *sems):
        in_sems = sems[:NBUF]
        out_sems = sems[NBUF:]
        wid = lax.axis_index("s") * NUM_SC_CORES + lax.axis_index("c")
        _repack_body(us_hbm, up_hbm, n_u, asm_v, in_sems, out_sems, wid)
        _repack_body(ms_hbm, mp_hbm, n_m, asm_v, in_sems, out_sems, wid)

    return repack_kernel(stg_m, stg_u)


def _sc_gather(mrow, urow, movie_packed, user_packed):
    batch = mrow.shape[0]
    b_per_w = batch // NUM_WORKERS
    mesh = plsc.VectorSubcoreMesh(core_axis_name="c", subcore_axis_name="s")

    @functools.partial(
        pl.kernel,
        mesh=mesh,
        out_type=(
            jax.ShapeDtypeStruct((batch, 128), jnp.float32),
            jax.ShapeDtypeStruct((batch, 128), jnp.float32),
        ),
        scratch_types=[
            pltpu.VMEM((b_per_w,), jnp.int32),
            pltpu.VMEM((b_per_w,), jnp.int32),
            pltpu.VMEM((CHUNK, 128), jnp.float32),
            pltpu.VMEM((CHUNK, 128), jnp.float32),
            pltpu.SemaphoreType.DMA,
            pltpu.SemaphoreType.DMA,
        ],
        compiler_params=pltpu.CompilerParams(use_tc_tiling_on_sc=False),
    )
    def gather_kernel(mt_hbm, ut_hbm, mi_hbm, ui_hbm, mo_hbm, uo_hbm,
                      mi_v, ui_v, mrows_v, urows_v, sem_m, sem_u):
        wid = lax.axis_index("s") * NUM_SC_CORES + lax.axis_index("c")
        base = wid * b_per_w
        pltpu.sync_copy(mi_hbm.at[pl.ds(base, b_per_w)], mi_v)
        pltpu.sync_copy(ui_hbm.at[pl.ds(base, b_per_w)], ui_v)

        @pl.loop(0, b_per_w, step=CHUNK)
        def _(c):
            cm = pltpu.async_copy(mt_hbm.at[mi_v.at[pl.ds(c, CHUNK)]],
                                  mrows_v, sem_m)
            cu = pltpu.async_copy(ut_hbm.at[ui_v.at[pl.ds(c, CHUNK)]],
                                  urows_v, sem_u)
            cm.wait()
            cu.wait()
            pltpu.sync_copy(mrows_v, mo_hbm.at[pl.ds(base + c, CHUNK)])
            pltpu.sync_copy(urows_v, uo_hbm.at[pl.ds(base + c, CHUNK)])

    return gather_kernel(movie_packed, user_packed, mrow, urow)


def _mlp_body(mc_ref, uc_ref, msub_ref, usub_ref, w0m_ref, w0u_ref, b0_ref,
              w1_ref, b1_ref, wo_ref, bo_ref, o_ref):
    col_mod = jax.lax.broadcasted_iota(jnp.int32, (1, 128), 1) % PACK
    mm = jnp.where(msub_ref[...] == col_mod, mc_ref[...], 0.0)
    uu = jnp.where(usub_ref[...] == col_mod, uc_ref[...], 0.0)
    h = (jnp.dot(mm, w0m_ref[...], preferred_element_type=jnp.float32)
         + jnp.dot(uu, w0u_ref[...], preferred_element_type=jnp.float32)
         + b0_ref[...])
    h = jnp.maximum(h, 0.0)
    h = jnp.dot(h, w1_ref[...], preferred_element_type=jnp.float32) + b1_ref[...]
    h = jnp.maximum(h, 0.0)
    o = jnp.dot(h, wo_ref[...], preferred_element_type=jnp.float32) + bo_ref[...]
    o_ref[...] = jax.nn.sigmoid(o)


def kernel(movie_id, user_id, movie_table, user_table, W0, b0, W1, b1, Wo, bo):
    batch = movie_id.shape[0]
    movie_id = movie_id.astype(jnp.int32)
    user_id = user_id.astype(jnp.int32)
    mt3 = movie_table.T.reshape(2, 8, movie_table.shape[0])
    ut3 = user_table.T.reshape(2, 8, user_table.shape[0])
    gm_pad = -(-movie_table.shape[0] // 128)
    gu_pad = -(-user_table.shape[0] // 128)
    stg_m, stg_u = _sc_linearize(mt3, ut3, gm_pad, gu_pad)
    stg_m5 = stg_m.reshape(2, gm_pad, 8, 16, 8)
    stg_u5 = stg_u.reshape(2, gu_pad, 8, 16, 8)
    mp, up = _sc_repack(stg_m5, stg_u5, movie_table.shape[0],
                        user_table.shape[0])
    mrow = movie_id >> 3
    urow = user_id >> 3
    msub = (movie_id & 7)[:, None]
    usub = (user_id & 7)[:, None]
    mc, uc = _sc_gather(mrow, urow, mp, up)
    w0m = jnp.repeat(W0[:, :EMBED_DIM].T, PACK, axis=0)
    w0u = jnp.repeat(W0[:, EMBED_DIM:].T, PACK, axis=0)
    out = pl.pallas_call(
        _mlp_body,
        out_shape=jax.ShapeDtypeStruct((batch, 1), jnp.float32),
    )(mc, uc, msub, usub, w0m, w0u, b0[None, :], W1.T, b1[None, :],
      Wo.T, bo[None, :])
    return out


# speedup vs baseline: 4.0584x; 4.0584x over previous
import functools

import jax
import jax.numpy as jnp
from jax import lax
from jax.experimental import pallas as pl
from jax.experimental.pallas import tpu as pltpu
from jax.experimental.pallas import tpu_sc as plsc

EMBED_DIM = 16
PACK = 8
NUM_SC_CORES = 2
NUM_SC_SUBCORES = 16
NUM_WORKERS = NUM_SC_CORES * NUM_SC_SUBCORES
NBUF = 2
CHUNK = 256


def _repack_block(x_ref, o_ref):
    x = x_ref[...]
    t8 = x.shape[2] // 8
    y = x.reshape(2, 8, t8, 8).transpose((2, 0, 1, 3))
    o_ref[...] = y.reshape(t8, 128)


def _tc_repack(tbl3, block_lanes):
    n = tbl3.shape[2]
    nblk = -(-n // block_lanes)
    return pl.pallas_call(
        _repack_block,
        grid=(nblk,),
        in_specs=[pl.BlockSpec((2, 8, block_lanes), lambda i: (0, 0, i))],
        out_specs=pl.BlockSpec((block_lanes // 8, 128), lambda i: (i, 0)),
        out_shape=jax.ShapeDtypeStruct((n // PACK, 128), jnp.float32),
    )(tbl3)


def _sc_gather(mrow, urow, movie_packed, user_packed):
    batch = mrow.shape[0]
    b_per_w = batch // NUM_WORKERS
    mesh = plsc.VectorSubcoreMesh(core_axis_name="c", subcore_axis_name="s")

    @functools.partial(
        pl.kernel,
        mesh=mesh,
        out_type=(
            jax.ShapeDtypeStruct((batch, 128), jnp.float32),
            jax.ShapeDtypeStruct((batch, 128), jnp.float32),
        ),
        scratch_types=[
            pltpu.VMEM((b_per_w,), jnp.int32),
            pltpu.VMEM((b_per_w,), jnp.int32),
            pltpu.VMEM((CHUNK, 128), jnp.float32),
            pltpu.VMEM((CHUNK, 128), jnp.float32),
            pltpu.SemaphoreType.DMA,
            pltpu.SemaphoreType.DMA,
        ],
        compiler_params=pltpu.CompilerParams(use_tc_tiling_on_sc=True),
    )
    def gather_kernel(mt_hbm, ut_hbm, mi_hbm, ui_hbm, mo_hbm, uo_hbm,
                      mi_v, ui_v, mrows_v, urows_v, sem_m, sem_u):
        wid = lax.axis_index("s") * NUM_SC_CORES + lax.axis_index("c")
        base = wid * b_per_w
        pltpu.sync_copy(mi_hbm.at[pl.ds(base, b_per_w)], mi_v)
        pltpu.sync_copy(ui_hbm.at[pl.ds(base, b_per_w)], ui_v)

        @pl.loop(0, b_per_w, step=CHUNK)
        def _(c):
            cm = pltpu.async_copy(mt_hbm.at[mi_v.at[pl.ds(c, CHUNK)]],
                                  mrows_v, sem_m)
            cu = pltpu.async_copy(ut_hbm.at[ui_v.at[pl.ds(c, CHUNK)]],
                                  urows_v, sem_u)
            cm.wait()
            cu.wait()
            pltpu.sync_copy(mrows_v, mo_hbm.at[pl.ds(base + c, CHUNK)])
            pltpu.sync_copy(urows_v, uo_hbm.at[pl.ds(base + c, CHUNK)])

    return gather_kernel(movie_packed, user_packed, mrow, urow)


def _mlp_body(mc_ref, uc_ref, msub_ref, usub_ref, w0m_ref, w0u_ref, b0_ref,
              w1_ref, b1_ref, wo_ref, bo_ref, o_ref):
    col_mod = jax.lax.broadcasted_iota(jnp.int32, (1, 128), 1) % PACK
    mm = jnp.where(msub_ref[...] == col_mod, mc_ref[...], 0.0)
    uu = jnp.where(usub_ref[...] == col_mod, uc_ref[...], 0.0)
    h = (jnp.dot(mm, w0m_ref[...], preferred_element_type=jnp.float32)
         + jnp.dot(uu, w0u_ref[...], preferred_element_type=jnp.float32)
         + b0_ref[...])
    h = jnp.maximum(h, 0.0)
    h = jnp.dot(h, w1_ref[...], preferred_element_type=jnp.float32) + b1_ref[...]
    h = jnp.maximum(h, 0.0)
    o = jnp.dot(h, wo_ref[...], preferred_element_type=jnp.float32) + bo_ref[...]
    o_ref[...] = jax.nn.sigmoid(o)


def kernel(movie_id, user_id, movie_table, user_table, W0, b0, W1, b1, Wo, bo):
    batch = movie_id.shape[0]
    movie_id = movie_id.astype(jnp.int32)
    user_id = user_id.astype(jnp.int32)
    mt3 = movie_table.T.reshape(2, 8, movie_table.shape[0])
    ut3 = user_table.T.reshape(2, 8, user_table.shape[0])
    mp = _tc_repack(mt3, 12800)
    up = _tc_repack(ut3, 12800)
    mrow = movie_id >> 3
    urow = user_id >> 3
    msub = (movie_id & 7)[:, None]
    usub = (user_id & 7)[:, None]
    mc, uc = _sc_gather(mrow, urow, mp, up)
    w0m = jnp.repeat(W0[:, :EMBED_DIM].T, PACK, axis=0)
    w0u = jnp.repeat(W0[:, EMBED_DIM:].T, PACK, axis=0)
    out = pl.pallas_call(
        _mlp_body,
        out_shape=jax.ShapeDtypeStruct((batch, 1), jnp.float32),
    )(mc, uc, msub, usub, w0m, w0u, b0[None, :], W1.T, b1[None, :],
      Wo.T, bo[None, :])
    return out


# speedup vs baseline: 6.3219x; 1.5577x over previous
import functools

import jax
import jax.numpy as jnp
from jax import lax
from jax.experimental import pallas as pl
from jax.experimental.pallas import tpu as pltpu
from jax.experimental.pallas import tpu_sc as plsc

EMBED_DIM = 16
NUM_SC_CORES = 2
NUM_SC_SUBCORES = 16
NUM_WORKERS = NUM_SC_CORES * NUM_SC_SUBCORES


def _sc_gather(movie_id, user_id, movie_table, user_table):
    batch = movie_id.shape[0]
    b_per_w = batch // NUM_WORKERS
    mesh = plsc.VectorSubcoreMesh(core_axis_name="c", subcore_axis_name="s")

    @functools.partial(
        pl.kernel,
        mesh=mesh,
        out_type=(
            jax.ShapeDtypeStruct((batch, EMBED_DIM), jnp.float32),
            jax.ShapeDtypeStruct((batch, EMBED_DIM), jnp.float32),
        ),
        scratch_types=[
            pltpu.VMEM((b_per_w,), jnp.int32),
            pltpu.VMEM((b_per_w,), jnp.int32),
            pltpu.VMEM((b_per_w, EMBED_DIM), jnp.float32),
            pltpu.VMEM((b_per_w, EMBED_DIM), jnp.float32),
            pltpu.SemaphoreType.DMA,
            pltpu.SemaphoreType.DMA,
        ],
        compiler_params=pltpu.CompilerParams(use_tc_tiling_on_sc=False),
    )
    def gather_kernel(mt_hbm, ut_hbm, mi_hbm, ui_hbm, mo_hbm, uo_hbm,
                      mi_v, ui_v, mrows_v, urows_v, sem_m, sem_u):
        wid = lax.axis_index("s") * NUM_SC_CORES + lax.axis_index("c")
        base = wid * b_per_w
        pltpu.sync_copy(mi_hbm.at[pl.ds(base, b_per_w)], mi_v)
        pltpu.sync_copy(ui_hbm.at[pl.ds(base, b_per_w)], ui_v)
        cm = pltpu.async_copy(mt_hbm.at[mi_v], mrows_v, sem_m)
        cu = pltpu.async_copy(ut_hbm.at[ui_v], urows_v, sem_u)
        cm.wait()
        cu.wait()
        pltpu.sync_copy(mrows_v, mo_hbm.at[pl.ds(base, b_per_w)])
        pltpu.sync_copy(urows_v, uo_hbm.at[pl.ds(base, b_per_w)])

    return gather_kernel(movie_table, user_table, movie_id, user_id)


def _mlp_body(me_ref, ue_ref, w0m_ref, w0u_ref, b0_ref, w1_ref, b1_ref,
              wo_ref, bo_ref, o_ref):
    h = (jnp.dot(me_ref[...], w0m_ref[...], preferred_element_type=jnp.float32)
         + jnp.dot(ue_ref[...], w0u_ref[...], preferred_element_type=jnp.float32)
         + b0_ref[...])
    h = jnp.maximum(h, 0.0)
    h = jnp.dot(h, w1_ref[...], preferred_element_type=jnp.float32) + b1_ref[...]
    h = jnp.maximum(h, 0.0)
    o = jnp.dot(h, wo_ref[...], preferred_element_type=jnp.float32) + bo_ref[...]
    o_ref[...] = jax.nn.sigmoid(o)


def kernel(movie_id, user_id, movie_table, user_table, W0, b0, W1, b1, Wo, bo):
    batch = movie_id.shape[0]
    movie_id = movie_id.astype(jnp.int32)
    user_id = user_id.astype(jnp.int32)
    me, ue = _sc_gather(movie_id, user_id, movie_table, user_table)
    w0m = W0[:, :EMBED_DIM].T
    w0u = W0[:, EMBED_DIM:].T
    out = pl.pallas_call(
        _mlp_body,
        out_shape=jax.ShapeDtypeStruct((batch, 1), jnp.float32),
    )(me, ue, w0m, w0u, b0[None, :], W1.T, b1[None, :], Wo.T, bo[None, :])
    return out
